# baseline (speedup 1.0000x reference)
import functools

import jax
import jax.numpy as jnp
import numpy as np
from jax import lax
from jax.experimental import pallas as pl
from jax.experimental.pallas import tpu as pltpu
from jax.experimental.pallas import tpu_sc as plsc

TWO_PI_E = 2.0 * np.pi * np.e
EPS = 1e-6

B = 16384
D = 64
G = 8
C = 2
NSEG = G * C
NC = 2
NS = 16
NW = NC * NS
RPW = B // NW
L = 16
W = 2 * D


def _sc_body(x_hbm, y_hbm, b_hbm, sq_out, n_out,
             x_v, xq_v, y_v, b_v, gid_v, cnt_v, ones_v, zb_v,
             sh_sq, sh_n, sem, sem_x, sem_y, sem_b):
    cid = lax.axis_index("c")
    sid = lax.axis_index("s")
    wid = cid * NS + sid
    base = wid * RPW

    cp_x = pltpu.async_copy(x_hbm.at[pl.ds(base, RPW)], x_v, sem_x)
    cp_y = pltpu.async_copy(y_hbm.at[pl.ds(base, RPW)], y_v, sem_y)
    cp_b = pltpu.async_copy(b_hbm.at[pl.ds(base, RPW)], b_v, sem_b)

    zeros = jnp.zeros((L,), jnp.float32)
    for r in range(NSEG):
        for k in range(W // L):
            zb_v[r, pl.ds(k * L, L)] = zeros
            ones_v[r, pl.ds(k * L, L)] = zeros
        cnt_v[pl.ds(r * L, L)] = zeros

    @pl.when(sid == 0)
    def _():
        pltpu.sync_copy(zb_v, sh_sq)
        pltpu.sync_copy(zb_v, sh_n)

    cp_y.wait()
    cp_b.wait()
    cc = jnp.full((L,), C, jnp.int32)
    for i in range(RPW // L):
        sl = pl.ds(i * L, L)
        gid_v[sl] = b_v[sl] * cc + y_v[sl]
    cp_x.wait()

    plsc.subcore_barrier()

    lane = lax.broadcasted_iota(jnp.int32, (L,), 0)
    onesf = jnp.full((L,), 1.0, jnp.float32)

    HALF = RPW // 2

    def make_chunk_body(base_row):
        def chunk_body(i, cnts):
            r0 = base_row + i * L
            q0 = i * L
            for rr in range(L):
                for k in range(D // L):
                    xk = x_v[r0 + rr, pl.ds(k * L, L)]
                    xq_v[q0 + rr, pl.ds(k * L, L)] = xk
                    xq_v[q0 + rr, pl.ds(D + k * L, L)] = xk * xk
            gvec = gid_v[pl.ds(r0, L)]
            pltpu.async_copy(xq_v.at[pl.ds(q0, L)], sh_sq.at[gvec], sem,
                             add=True)
            return tuple(cnts[s] + jnp.where(gvec == s, 1.0, 0.0)
                         for s in range(NSEG))
        return chunk_body

    def drain(i, carry):
        pltpu.make_async_copy(sq_out, xq_v.at[pl.ds(i * NC * NSEG, NC * NSEG)],
                              sem).wait()
        return carry

    cnts0 = tuple(jnp.zeros((L,), jnp.float32) for _ in range(NSEG))
    cnts = lax.fori_loop(0, HALF // L, make_chunk_body(0), cnts0, unroll=2)
    lax.fori_loop(0, HALF // (NC * NSEG), drain, 0)
    cnts = lax.fori_loop(0, HALF // L, make_chunk_body(HALF), cnts, unroll=2)
    lax.fori_loop(0, HALF // (NC * NSEG), drain, 0)

    for s in range(NSEG):
        ones_v[s, pl.ds(0, L)] = cnts[s]
    pltpu.sync_copy(ones_v, sh_n.at[lane], add=True)

    plsc.subcore_barrier()

    @pl.when(sid == 0)
    def _():
        pltpu.sync_copy(sh_sq, sq_out.at[pl.ds(cid * NSEG, NSEG)])
        pltpu.sync_copy(sh_n, n_out.at[pl.ds(cid * NSEG, NSEG)])


@functools.partial(
    pl.kernel,
    out_type=(
        jax.ShapeDtypeStruct((NC * NSEG, W), jnp.float32),
        jax.ShapeDtypeStruct((NC * NSEG, W), jnp.float32),
    ),
    mesh=plsc.VectorSubcoreMesh(core_axis_name="c", subcore_axis_name="s"),
    scratch_types=[
        pltpu.VMEM((RPW, D), jnp.float32),
        pltpu.VMEM((RPW // 2, W), jnp.float32),
        pltpu.VMEM((RPW,), jnp.int32),
        pltpu.VMEM((RPW,), jnp.int32),
        pltpu.VMEM((RPW,), jnp.int32),
        pltpu.VMEM((NSEG * L,), jnp.float32),
        pltpu.VMEM((NSEG, W), jnp.float32),
        pltpu.VMEM((NSEG, W), jnp.float32),
        pltpu.VMEM_SHARED((NSEG, W), jnp.float32),
        pltpu.VMEM_SHARED((NSEG, W), jnp.float32),
        pltpu.SemaphoreType.DMA,
        pltpu.SemaphoreType.DMA,
        pltpu.SemaphoreType.DMA,
        pltpu.SemaphoreType.DMA,
    ],
)
def _sc_partials(*args):
    _sc_body(*args)


def _fin_body(sq_ref, n_ref, out_ref):
    SQ = jnp.sum(sq_ref[...], axis=0)
    S_gc = SQ[:, 0:D]
    Q_gc = SQ[:, D:W]
    N_gc = jnp.sum(jnp.sum(n_ref[...], axis=0)[:, 0:L],
                   axis=1, keepdims=True)

    Nc = jnp.maximum(N_gc, 1.0)
    mean_gc = S_gc / Nc
    var_gc = jnp.maximum(Q_gc / Nc - mean_gc * mean_gc, EPS)
    H_gc = 0.5 * jnp.sum(jnp.log(TWO_PI_E * var_gc), axis=1, keepdims=True)

    pi = lax.broadcasted_iota(jnp.int32, (G, NSEG), 0)
    pj = lax.broadcasted_iota(jnp.int32, (G, NSEG), 1)
    P = jnp.where(pj // C == pi, 1.0, 0.0).astype(jnp.float32)

    N_g = jnp.dot(P, N_gc, preferred_element_type=jnp.float32,
                  precision=lax.Precision.HIGHEST)
    S_g = jnp.dot(P, S_gc, preferred_element_type=jnp.float32,
                  precision=lax.Precision.HIGHEST)
    Q_g = jnp.dot(P, Q_gc, preferred_element_type=jnp.float32,
                  precision=lax.Precision.HIGHEST)

    Ng = jnp.maximum(N_g, 1.0)
    mean_g = S_g / Ng
    var_g = jnp.maximum(Q_g / Ng - mean_g * mean_g, EPS)
    H_marg = 0.5 * jnp.sum(jnp.log(TWO_PI_E * var_g), axis=1, keepdims=True)

    Ng_rows = jnp.dot(P.T, Ng, preferred_element_type=jnp.float32,
                      precision=lax.Precision.HIGHEST)
    p_gc = N_gc / Ng_rows
    H_cond = jnp.dot(P, p_gc * H_gc, preferred_element_type=jnp.float32,
                     precision=lax.Precision.HIGHEST)

    mi = H_marg - H_cond
    out_ref[...] = jnp.broadcast_to(-jnp.sum(mi) * (1.0 / G), (1, 1))


def kernel(X, y, batch, batch_size, n_classes, samples_set_per_batch):
    X = X.astype(jnp.float32)
    y = y.astype(jnp.int32)
    batch = batch.astype(jnp.int32)
    sq_p, n_p = _sc_partials(X, y, batch)
    out = pl.pallas_call(
        _fin_body,
        out_shape=jax.ShapeDtypeStruct((1, 1), jnp.float32),
    )(sq_p.reshape(NC, NSEG, W), n_p.reshape(NC, NSEG, W))
    return out[0, 0]


# speedup vs baseline: 1.0700x; 1.0700x over previous
import functools

import jax
import jax.numpy as jnp
import numpy as np
from jax import lax
from jax.experimental import pallas as pl
from jax.experimental.pallas import tpu as pltpu
from jax.experimental.pallas import tpu_sc as plsc

TWO_PI_E = 2.0 * np.pi * np.e
EPS = 1e-6

B = 16384
D = 64
G = 8
C = 2
NSEG = G * C
NC = 2
NS = 16
NW = NC * NS
RPW = B // NW
L = 16
W = 2 * D


def _sc_body(x_hbm, y_hbm, b_hbm, sq_out, n_out,
             x_v, xq_v, y_v, b_v, gid_v, cnt_v, ones_v, zb_v,
             sh_sq, sh_n, sem, sem_x, sem_y, sem_b):
    cid = lax.axis_index("c")
    sid = lax.axis_index("s")
    wid = cid * NS + sid
    base = wid * RPW

    cp_x = pltpu.async_copy(x_hbm.at[pl.ds(base, RPW)], x_v, sem_x)
    cp_y = pltpu.async_copy(y_hbm.at[pl.ds(base, RPW)], y_v, sem_y)
    cp_b = pltpu.async_copy(b_hbm.at[pl.ds(base, RPW)], b_v, sem_b)

    zeros = jnp.zeros((L,), jnp.float32)
    for r in range(NSEG):
        for k in range(W // L):
            zb_v[r, pl.ds(k * L, L)] = zeros
            ones_v[r, pl.ds(k * L, L)] = zeros
        cnt_v[pl.ds(r * L, L)] = zeros

    @pl.when(sid == 0)
    def _():
        pltpu.sync_copy(zb_v, sh_sq)
        pltpu.sync_copy(zb_v, sh_n)

    cp_y.wait()
    cp_b.wait()
    cc = jnp.full((L,), C, jnp.int32)
    for i in range(RPW // L):
        sl = pl.ds(i * L, L)
        gid_v[sl] = b_v[sl] * cc + y_v[sl]
    cp_x.wait()

    plsc.subcore_barrier()

    lane = lax.broadcasted_iota(jnp.int32, (L,), 0)
    onesf = jnp.full((L,), 1.0, jnp.float32)

    HALF = RPW // 2

    def make_chunk_body(base_row):
        def chunk_body(i, cnts):
            r0 = base_row + i * L
            q0 = i * L
            for rr in range(L):
                for k in range(D // L):
                    xk = x_v[r0 + rr, pl.ds(k * L, L)]
                    xq_v[q0 + rr, pl.ds(k * L, L)] = xk
                    xq_v[q0 + rr, pl.ds(D + k * L, L)] = xk * xk
            gvec = gid_v[pl.ds(r0, L)]
            pltpu.async_copy(xq_v.at[pl.ds(q0, L)], sh_sq.at[gvec], sem,
                             add=True)
            return tuple(cnts[s] + jnp.where(gvec == s, 1.0, 0.0)
                         for s in range(NSEG))
        return chunk_body

    def drain(i, carry):
        pltpu.make_async_copy(sq_out, xq_v.at[pl.ds(i * NC * NSEG, NC * NSEG)],
                              sem).wait()
        return carry

    cnts0 = tuple(jnp.zeros((L,), jnp.float32) for _ in range(NSEG))
    cnts = lax.fori_loop(0, HALF // L, make_chunk_body(0), cnts0)
    lax.fori_loop(0, HALF // (NC * NSEG), drain, 0)
    cnts = lax.fori_loop(0, HALF // L, make_chunk_body(HALF), cnts)
    lax.fori_loop(0, HALF // (NC * NSEG), drain, 0)

    for s in range(NSEG):
        ones_v[s, pl.ds(0, L)] = cnts[s]
    pltpu.sync_copy(ones_v, sh_n.at[lane], add=True)

    plsc.subcore_barrier()

    @pl.when(sid == 0)
    def _():
        pltpu.sync_copy(sh_sq, sq_out.at[pl.ds(cid * NSEG, NSEG)])
        pltpu.sync_copy(sh_n, n_out.at[pl.ds(cid * NSEG, NSEG)])


@functools.partial(
    pl.kernel,
    out_type=(
        jax.ShapeDtypeStruct((NC * NSEG, W), jnp.float32),
        jax.ShapeDtypeStruct((NC * NSEG, W), jnp.float32),
    ),
    mesh=plsc.VectorSubcoreMesh(core_axis_name="c", subcore_axis_name="s"),
    scratch_types=[
        pltpu.VMEM((RPW, D), jnp.float32),
        pltpu.VMEM((RPW // 2, W), jnp.float32),
        pltpu.VMEM((RPW,), jnp.int32),
        pltpu.VMEM((RPW,), jnp.int32),
        pltpu.VMEM((RPW,), jnp.int32),
        pltpu.VMEM((NSEG * L,), jnp.float32),
        pltpu.VMEM((NSEG, W), jnp.float32),
        pltpu.VMEM((NSEG, W), jnp.float32),
        pltpu.VMEM_SHARED((NSEG, W), jnp.float32),
        pltpu.VMEM_SHARED((NSEG, W), jnp.float32),
        pltpu.SemaphoreType.DMA,
        pltpu.SemaphoreType.DMA,
        pltpu.SemaphoreType.DMA,
        pltpu.SemaphoreType.DMA,
    ],
)
def _sc_partials(*args):
    _sc_body(*args)


def _fin_body(sq_ref, n_ref, out_ref):
    SQ = jnp.sum(sq_ref[...], axis=0)
    S_gc = SQ[:, 0:D]
    Q_gc = SQ[:, D:W]
    N_gc = jnp.sum(jnp.sum(n_ref[...], axis=0)[:, 0:L],
                   axis=1, keepdims=True)

    Nc = jnp.maximum(N_gc, 1.0)
    mean_gc = S_gc / Nc
    var_gc = jnp.maximum(Q_gc / Nc - mean_gc * mean_gc, EPS)
    H_gc = 0.5 * jnp.sum(jnp.log(TWO_PI_E * var_gc), axis=1, keepdims=True)

    pi = lax.broadcasted_iota(jnp.int32, (G, NSEG), 0)
    pj = lax.broadcasted_iota(jnp.int32, (G, NSEG), 1)
    P = jnp.where(pj // C == pi, 1.0, 0.0).astype(jnp.float32)

    N_g = jnp.dot(P, N_gc, preferred_element_type=jnp.float32,
                  precision=lax.Precision.HIGHEST)
    S_g = jnp.dot(P, S_gc, preferred_element_type=jnp.float32,
                  precision=lax.Precision.HIGHEST)
    Q_g = jnp.dot(P, Q_gc, preferred_element_type=jnp.float32,
                  precision=lax.Precision.HIGHEST)

    Ng = jnp.maximum(N_g, 1.0)
    mean_g = S_g / Ng
    var_g = jnp.maximum(Q_g / Ng - mean_g * mean_g, EPS)
    H_marg = 0.5 * jnp.sum(jnp.log(TWO_PI_E * var_g), axis=1, keepdims=True)

    Ng_rows = jnp.dot(P.T, Ng, preferred_element_type=jnp.float32,
                      precision=lax.Precision.HIGHEST)
    p_gc = N_gc / Ng_rows
    H_cond = jnp.dot(P, p_gc * H_gc, preferred_element_type=jnp.float32,
                     precision=lax.Precision.HIGHEST)

    mi = H_marg - H_cond
    out_ref[...] = jnp.broadcast_to(-jnp.sum(mi) * (1.0 / G), (1, 1))


def kernel(X, y, batch, batch_size, n_classes, samples_set_per_batch):
    X = X.astype(jnp.float32)
    y = y.astype(jnp.int32)
    batch = batch.astype(jnp.int32)
    sq_p, n_p = _sc_partials(X, y, batch)
    out = pl.pallas_call(
        _fin_body,
        out_shape=jax.ShapeDtypeStruct((1, 1), jnp.float32),
    )(sq_p.reshape(NC, NSEG, W), n_p.reshape(NC, NSEG, W))
    return out[0, 0]
